# Initial kernel scaffold; baseline (speedup 1.0000x reference)
import jax
import jax.numpy as jnp
from jax.experimental import pallas as pl


def kernel(x_user, x_item, edge_index_ui, edge_index_iu, W_buys, b_buys, W_rev, b_rev):
    raise NotImplementedError("write your pallas kernel here")


# speedup vs baseline: 6.2381x; 6.2381x over previous
import functools

import jax
import jax.numpy as jnp
from jax import lax
from jax.experimental import pallas as pl
from jax.experimental.pallas import tpu as pltpu
from jax.experimental.pallas import tpu_sc as plsc

N = 50000
D = 128
E = 300000

NC = 2
NS = 16
LANES = 16

HP = 50176
HTILE = HP // NS
EPA_TILE = 10240
EPA = EPA_TILE * NC * NS

EPC_TILE = 20480
EPC = EPC_TILE * NS
WIN = 2048
CH = 12544
ACC_ROWS = 12672
GB = 128
CAP = WIN + 2 * GB
BIG = 1 << 30

USER_BASE = 0
ITEM_BASE = 52000
AGG_ROWS = ITEM_BASE + 4 * CH

BLK = 2000


def _hist_body(ids_hbm, zeros_hbm, out_hbm, h0, h1, h2, h3, iwin, didx, ones_v, stage):
    cid = lax.axis_index("c")
    sid = lax.axis_index("s")
    w32 = cid * NS + sid
    hists = (h0, h1, h2, h3)

    one16 = jnp.ones((LANES,), jnp.float32)
    for j in range(8):
        ones_v[pl.ds(j * LANES, LANES)] = one16

    pltpu.sync_copy(zeros_hbm, stage)
    for h in hists:
        pltpu.sync_copy(stage, h.at[pl.ds(sid * HTILE, 2048)])
        pltpu.sync_copy(stage.at[pl.ds(0, HTILE - 2048)],
                        h.at[pl.ds(sid * HTILE + 2048, HTILE - 2048)])
    plsc.subcore_barrier()

    for hi, h in enumerate(hists):
        pltpu.sync_copy(ids_hbm.at[pl.ds(hi * EPA + w32 * EPA_TILE, EPA_TILE)], iwin)

        def _batch(b, _, h=h):
            for j in range(8):
                didx[pl.ds(j * LANES, LANES)] = iwin[pl.ds(b * GB + j * LANES, LANES)]
            pltpu.sync_copy(ones_v, h.at[didx], add=True)
            return _

        lax.fori_loop(0, EPA_TILE // GB, _batch, jnp.int32(0))
    plsc.subcore_barrier()

    for hi, h in enumerate(hists):
        obase = cid * (4 * HP) + hi * HP + sid * HTILE
        pltpu.sync_copy(h.at[pl.ds(sid * HTILE, 2048)], stage)
        pltpu.sync_copy(stage, out_hbm.at[pl.ds(obase, 2048)])
        pltpu.sync_copy(h.at[pl.ds(sid * HTILE + 2048, HTILE - 2048)],
                        stage.at[pl.ds(0, HTILE - 2048)])
        pltpu.sync_copy(stage.at[pl.ds(0, HTILE - 2048)],
                        out_hbm.at[pl.ds(obase + 2048, HTILE - 2048)])


_hist_kernel = functools.partial(
    pl.kernel,
    compiler_params=pltpu.CompilerParams(needs_layout_passes=False),
    out_type=jax.ShapeDtypeStruct((NC * 4 * HP,), jnp.float32),
    mesh=plsc.VectorSubcoreMesh(core_axis_name="c", subcore_axis_name="s"),
    scratch_types=[
        pltpu.VMEM_SHARED((HP,), jnp.float32),
        pltpu.VMEM_SHARED((HP,), jnp.float32),
        pltpu.VMEM_SHARED((HP,), jnp.float32),
        pltpu.VMEM_SHARED((HP,), jnp.float32),
        pltpu.VMEM((EPA_TILE,), jnp.int32),
        pltpu.VMEM((GB,), jnp.int32),
        pltpu.VMEM((GB,), jnp.float32),
        pltpu.VMEM((2048,), jnp.float32),
    ],
)(_hist_body)


def _agg_body(xs0_hbm, xs1_hbm, s0_hbm, d0_hbm, s1_hbm, d1_hbm, zrows_hbm,
              agg_hbm, acc, swin, dwin, srcflat, dstflat, sidx, didx, rows):
    cid = lax.axis_index("c")
    sid = lax.axis_index("s")
    li = lax.broadcasted_iota(jnp.int32, (LANES,), 0)

    convs = ((xs0_hbm, s0_hbm, d0_hbm, USER_BASE), (xs1_hbm, s1_hbm, d1_hbm, ITEM_BASE))
    for xs_hbm, src_hbm, dst_hbm, out_ntype_base in convs:
        for ck in range(2):
            chunk = 2 * ck + cid
            base = chunk * CH

            pltpu.sync_copy(zrows_hbm, rows)
            z0 = sid * (ACC_ROWS // NS)
            for j in range(6):
                pltpu.sync_copy(rows, acc.at[pl.ds(z0 + j * 128, 128)])
            pltpu.sync_copy(rows.at[pl.ds(0, 24)], acc.at[pl.ds(z0 + 768, 24)])
            plsc.subcore_barrier()

            def _fire(b, carry):
                for j in range(8):
                    sidx[pl.ds(j * LANES, LANES)] = srcflat[pl.ds(b * GB + j * LANES, LANES)]
                    didx[pl.ds(j * LANES, LANES)] = dstflat[pl.ds(b * GB + j * LANES, LANES)]
                pltpu.sync_copy(xs_hbm.at[sidx], rows)
                pltpu.sync_copy(rows, acc.at[didx], add=True)
                return carry

            cnt = jnp.int32(0)
            for w in range(EPC_TILE // WIN):
                eoff = sid * EPC_TILE + w * WIN
                pltpu.sync_copy(src_hbm.at[pl.ds(eoff, WIN)], swin)
                pltpu.sync_copy(dst_hbm.at[pl.ds(eoff, WIN)], dwin)

                def _compact(i, c):
                    d = dwin[pl.ds(i * LANES, LANES)]
                    s = swin[pl.ds(i * LANES, LANES)]
                    m = (d >= base) & (d < base + CH)
                    mi = m.astype(jnp.int32)
                    pos = c + plsc.cumsum(mi) - mi
                    plsc.store_scatter(srcflat, [pos], s, mask=m)
                    plsc.store_scatter(dstflat, [pos], d - base, mask=m)
                    return c + jnp.sum(mi)

                cnt = lax.fori_loop(0, WIN // LANES, _compact, cnt)
                nb = cnt // GB
                lax.fori_loop(0, nb, _fire, jnp.int32(0))
                for j in range(8):
                    sv = srcflat[pl.ds(nb * GB + j * LANES, LANES)]
                    dv = dstflat[pl.ds(nb * GB + j * LANES, LANES)]
                    srcflat[pl.ds(j * LANES, LANES)] = sv
                    dstflat[pl.ds(j * LANES, LANES)] = dv
                cnt = cnt - nb * GB

            for j in range(8):
                srcflat[pl.ds(cnt + j * LANES, LANES)] = sid * 128 + j * LANES + li
                dstflat[pl.ds(cnt + j * LANES, LANES)] = CH + li

            @pl.when(cnt > 0)
            def _():
                _fire(jnp.int32(0), jnp.int32(0))

            plsc.subcore_barrier()

            r0 = sid * (CH // NS)
            o0 = out_ntype_base + base + r0
            for j in range(6):
                pltpu.sync_copy(acc.at[pl.ds(r0 + j * 128, 128)], rows)
                pltpu.sync_copy(rows, agg_hbm.at[pl.ds(o0 + j * 128, 128)])
            pltpu.sync_copy(acc.at[pl.ds(r0 + 768, 16)], rows.at[pl.ds(0, 16)])
            pltpu.sync_copy(rows.at[pl.ds(0, 16)], agg_hbm.at[pl.ds(o0 + 768, 16)])
            plsc.subcore_barrier()


_agg_kernel = functools.partial(
    pl.kernel,
    compiler_params=pltpu.CompilerParams(needs_layout_passes=False),
    out_type=jax.ShapeDtypeStruct((AGG_ROWS, D), jnp.float32),
    mesh=plsc.VectorSubcoreMesh(core_axis_name="c", subcore_axis_name="s"),
    scratch_types=[
        pltpu.VMEM_SHARED((ACC_ROWS, D), jnp.float32),
        pltpu.VMEM((WIN,), jnp.int32),
        pltpu.VMEM((WIN,), jnp.int32),
        pltpu.VMEM((CAP,), jnp.int32),
        pltpu.VMEM((CAP,), jnp.int32),
        pltpu.VMEM((GB,), jnp.int32),
        pltpu.VMEM((GB,), jnp.int32),
        pltpu.VMEM((GB, D), jnp.float32),
    ],
)(_agg_body)


def _prescale_body(xu, xi, h, xsu, xsi):
    d0 = h[:, 0] + h[:, 1]
    xsu[...] = xu[...] * lax.rsqrt(jnp.clip(d0, 1.0, None))[:, None]
    d2 = h[:, 4] + h[:, 5]
    xsi[...] = xi[...] * lax.rsqrt(jnp.clip(d2, 1.0, None))[:, None]


def _prescale(x_user, x_item, hists_t):
    return pl.pallas_call(
        _prescale_body,
        grid=(N // BLK,),
        in_specs=[
            pl.BlockSpec((BLK, D), lambda i: (i, 0)),
            pl.BlockSpec((BLK, D), lambda i: (i, 0)),
            pl.BlockSpec((BLK, 8), lambda i: (i, 0)),
        ],
        out_specs=[
            pl.BlockSpec((BLK, D), lambda i: (i, 0)),
            pl.BlockSpec((BLK, D), lambda i: (i, 0)),
        ],
        out_shape=[
            jax.ShapeDtypeStruct((N, D), jnp.float32),
            jax.ShapeDtypeStruct((N, D), jnp.float32),
        ],
    )(x_user, x_item, hists_t)


def _out_body(agg, hd, w, b, out):
    nb = N // BLK
    is_user = pl.program_id(0) < nb
    h = hd[...]
    deg = jnp.where(is_user, h[:, 6] + h[:, 7], h[:, 2] + h[:, 3])
    nd = lax.rsqrt(jnp.clip(deg, 1.0, None))
    out[...] = (
        jnp.dot(agg[...] * nd[:, None], w[0], preferred_element_type=jnp.float32)
        + b[0, 0][None, :]
    )


def _finalize(agg, hists_t, w_st, b_st):
    nb = N // BLK
    return pl.pallas_call(
        _out_body,
        grid=(2 * nb,),
        in_specs=[
            pl.BlockSpec((BLK, D), lambda i: (jnp.where(i < nb, i, i + 1), 0)),
            pl.BlockSpec((BLK, 8), lambda i: (lax.rem(i, nb), 0)),
            pl.BlockSpec((1, D, D), lambda i: (i // nb, 0, 0)),
            pl.BlockSpec((1, 1, D), lambda i: (i // nb, 0, 0)),
        ],
        out_specs=pl.BlockSpec((BLK, D), lambda i: (i, 0)),
        out_shape=jax.ShapeDtypeStruct((2 * N, D), jnp.float32),
    )(agg, hists_t, w_st, b_st)


def kernel(x_user, x_item, edge_index_ui, edge_index_iu, W_buys, b_buys, W_rev, b_rev):
    su, du = edge_index_ui[0], edge_index_ui[1]
    si, di = edge_index_iu[0], edge_index_iu[1]

    pad_a = EPA - E
    sent = (N + (jnp.arange(pad_a, dtype=jnp.int32) % 128)).astype(jnp.int32)
    ids_a = jnp.concatenate([
        jnp.concatenate([su, sent]),
        jnp.concatenate([du, sent]),
        jnp.concatenate([si, sent]),
        jnp.concatenate([di, sent]),
    ])

    pad_c = EPC - E
    zpad = jnp.zeros((pad_c,), jnp.int32)
    bpad = jnp.full((pad_c,), BIG, jnp.int32)
    s0 = jnp.concatenate([si, zpad])
    d0 = jnp.concatenate([di, bpad])
    s1 = jnp.concatenate([su, zpad])
    d1 = jnp.concatenate([du, bpad])

    zeros_a = jnp.zeros((2048,), jnp.float32)
    zrows = jnp.zeros((128, D), jnp.float32)

    hist_p = _hist_kernel(ids_a, zeros_a).reshape(NC, 4, HP)
    hists_t = jnp.transpose(hist_p, (1, 0, 2))[:, :, :N].reshape(8, N).T

    xs_user, xs_item = _prescale(x_user, x_item, hists_t)
    agg = _agg_kernel(xs_item, xs_user, s0, d0, s1, d1, zrows)

    w_st = jnp.stack([W_rev, W_buys])
    b_st = jnp.stack([b_rev, b_buys])[:, None, :]
    return _finalize(agg, hists_t, w_st, b_st)


# speedup vs baseline: 6.8004x; 1.0901x over previous
import functools

import jax
import jax.numpy as jnp
from jax import lax
from jax.experimental import pallas as pl
from jax.experimental.pallas import tpu as pltpu
from jax.experimental.pallas import tpu_sc as plsc

N = 50000
D = 128
E = 300000

NC = 2
NS = 16
LANES = 16

HP = 50176
HTILE = HP // NS
EPA_TILE = 10240
HGB = 128
EPA = EPA_TILE * NC * NS

EPC_TILE = 20480
EPC = EPC_TILE * NS
WIN = 2048
CH = 12544
ACC_ROWS = 12672
GB = 64
CAP = WIN + 2 * GB
BIG = 1 << 30

USER_BASE = 0
ITEM_BASE = 52000
AGG_ROWS = ITEM_BASE + 4 * CH

BLK = 2000


def _hist_body(ids_hbm, zeros_hbm, out_hbm, h0, h1, h2, h3, iwin, didx, ones_v, stage):
    cid = lax.axis_index("c")
    sid = lax.axis_index("s")
    w32 = cid * NS + sid
    hists = (h0, h1, h2, h3)

    one16 = jnp.ones((LANES,), jnp.float32)
    for j in range(8):
        ones_v[pl.ds(j * LANES, LANES)] = one16

    pltpu.sync_copy(zeros_hbm, stage)
    for h in hists:
        pltpu.sync_copy(stage, h.at[pl.ds(sid * HTILE, 2048)])
        pltpu.sync_copy(stage.at[pl.ds(0, HTILE - 2048)],
                        h.at[pl.ds(sid * HTILE + 2048, HTILE - 2048)])
    plsc.subcore_barrier()

    for hi, h in enumerate(hists):
        pltpu.sync_copy(ids_hbm.at[pl.ds(hi * EPA + w32 * EPA_TILE, EPA_TILE)], iwin)

        def _batch(b, _, h=h):
            for j in range(8):
                didx[pl.ds(j * LANES, LANES)] = iwin[pl.ds(b * HGB + j * LANES, LANES)]
            pltpu.sync_copy(ones_v, h.at[didx], add=True)
            return _

        lax.fori_loop(0, EPA_TILE // HGB, _batch, jnp.int32(0))
    plsc.subcore_barrier()

    for hi, h in enumerate(hists):
        obase = cid * (4 * HP) + hi * HP + sid * HTILE
        pltpu.sync_copy(h.at[pl.ds(sid * HTILE, 2048)], stage)
        pltpu.sync_copy(stage, out_hbm.at[pl.ds(obase, 2048)])
        pltpu.sync_copy(h.at[pl.ds(sid * HTILE + 2048, HTILE - 2048)],
                        stage.at[pl.ds(0, HTILE - 2048)])
        pltpu.sync_copy(stage.at[pl.ds(0, HTILE - 2048)],
                        out_hbm.at[pl.ds(obase + 2048, HTILE - 2048)])


_hist_kernel = functools.partial(
    pl.kernel,
    compiler_params=pltpu.CompilerParams(needs_layout_passes=False),
    out_type=jax.ShapeDtypeStruct((NC * 4 * HP,), jnp.float32),
    mesh=plsc.VectorSubcoreMesh(core_axis_name="c", subcore_axis_name="s"),
    scratch_types=[
        pltpu.VMEM_SHARED((HP,), jnp.float32),
        pltpu.VMEM_SHARED((HP,), jnp.float32),
        pltpu.VMEM_SHARED((HP,), jnp.float32),
        pltpu.VMEM_SHARED((HP,), jnp.float32),
        pltpu.VMEM((EPA_TILE,), jnp.int32),
        pltpu.VMEM((HGB,), jnp.int32),
        pltpu.VMEM((HGB,), jnp.float32),
        pltpu.VMEM((2048,), jnp.float32),
    ],
)(_hist_body)


def _agg_body(xs0_hbm, xs1_hbm, s0_hbm, d0_hbm, s1_hbm, d1_hbm, zrows_hbm,
              agg_hbm, acc, swa, dwa, swb, dwb, srcflat, dstflat,
              sidx0, didx0, sidx1, didx1, rows0, rows1,
              sem_g0, sem_g1, sem_wsa, sem_wda, sem_wsb, sem_wdb):
    cid = lax.axis_index("c")
    sid = lax.axis_index("s")
    li = lax.broadcasted_iota(jnp.int32, (LANES,), 0)
    nwin = EPC_TILE // WIN
    wbufs = [(swa, dwa, sem_wsa, sem_wda), (swb, dwb, sem_wsb, sem_wdb)]

    convs = ((xs0_hbm, s0_hbm, d0_hbm, USER_BASE), (xs1_hbm, s1_hbm, d1_hbm, ITEM_BASE))
    for xs_hbm, src_hbm, dst_hbm, out_ntype_base in convs:
        def _do_chunk(ck, carry, xs_hbm=xs_hbm, src_hbm=src_hbm,
                      dst_hbm=dst_hbm, out_ntype_base=out_ntype_base):
            chunk = 2 * ck + cid
            base = chunk * CH

            pltpu.sync_copy(zrows_hbm, rows0)
            z0 = sid * (ACC_ROWS // NS)

            def _zero(j, _):
                pltpu.sync_copy(rows0, acc.at[pl.ds(z0 + j * GB, GB)])
                return _

            lax.fori_loop(0, 12, _zero, jnp.int32(0))
            pltpu.sync_copy(rows0.at[pl.ds(0, 24)], acc.at[pl.ds(z0 + 768, 24)])
            plsc.subcore_barrier()

            def _start_gather(b, si, rb, sem):
                for j in range(GB // LANES):
                    si[pl.ds(j * LANES, LANES)] = srcflat[pl.ds(b * GB + j * LANES, LANES)]
                return pltpu.async_copy(xs_hbm.at[si], rb, sem)

            def _scatter(b, di, rb):
                for j in range(GB // LANES):
                    di[pl.ds(j * LANES, LANES)] = dstflat[pl.ds(b * GB + j * LANES, LANES)]
                pltpu.sync_copy(rb, acc.at[di], add=True)

            def _fire2(k, nb):
                b0 = 2 * k
                g0 = _start_gather(b0, sidx0, rows0, sem_g0)

                @pl.when(b0 + 1 < nb)
                def _():
                    g1 = _start_gather(b0 + 1, sidx1, rows1, sem_g1)
                    g1.wait()

                g0.wait()
                _scatter(b0, didx0, rows0)

                @pl.when(b0 + 1 < nb)
                def _():
                    _scatter(b0 + 1, didx1, rows1)

                return nb

            sw0, dw0, sws0, swd0 = wbufs[0]
            c_s = pltpu.async_copy(src_hbm.at[pl.ds(sid * EPC_TILE, WIN)], sw0, sws0)
            c_d = pltpu.async_copy(dst_hbm.at[pl.ds(sid * EPC_TILE, WIN)], dw0, swd0)
            pend = (c_s, c_d)
            cnt = jnp.int32(0)
            for w in range(nwin):
                sw, dw, _, _ = wbufs[w % 2]
                pend[0].wait()
                pend[1].wait()
                if w + 1 < nwin:
                    swn, dwn, semsn, semdn = wbufs[(w + 1) % 2]
                    eoff = sid * EPC_TILE + (w + 1) * WIN
                    c_s = pltpu.async_copy(src_hbm.at[pl.ds(eoff, WIN)], swn, semsn)
                    c_d = pltpu.async_copy(dst_hbm.at[pl.ds(eoff, WIN)], dwn, semdn)
                    pend = (c_s, c_d)

                def _compact(i, c, dw=dw, sw=sw):
                    d = dw[pl.ds(i * LANES, LANES)]
                    s = sw[pl.ds(i * LANES, LANES)]
                    m = (d >= base) & (d < base + CH)
                    mi = m.astype(jnp.int32)
                    pos = c + plsc.cumsum(mi) - mi
                    plsc.store_scatter(srcflat, [pos], s, mask=m)
                    plsc.store_scatter(dstflat, [pos], d - base, mask=m)
                    return c + jnp.sum(mi)

                cnt = lax.fori_loop(0, WIN // LANES, _compact, cnt)
                nb = cnt // GB
                lax.fori_loop(0, (nb + 1) // 2, _fire2, nb)
                for j in range(GB // LANES):
                    sv = srcflat[pl.ds(nb * GB + j * LANES, LANES)]
                    dv = dstflat[pl.ds(nb * GB + j * LANES, LANES)]
                    srcflat[pl.ds(j * LANES, LANES)] = sv
                    dstflat[pl.ds(j * LANES, LANES)] = dv
                cnt = cnt - nb * GB

            for j in range(GB // LANES):
                srcflat[pl.ds(cnt + j * LANES, LANES)] = sid * 128 + j * LANES + li
                dstflat[pl.ds(cnt + j * LANES, LANES)] = CH + li

            @pl.when(cnt > 0)
            def _():
                _start_gather(jnp.int32(0), sidx0, rows0, sem_g0).wait()
                _scatter(jnp.int32(0), didx0, rows0)

            plsc.subcore_barrier()

            r0 = sid * (CH // NS)
            o0 = out_ntype_base + base + r0
            descs = [None, None]
            for j in range(12):
                rb = (rows0, rows1)[j % 2]
                if descs[j % 2] is not None:
                    descs[j % 2].wait()
                pltpu.sync_copy(acc.at[pl.ds(r0 + j * GB, GB)], rb)
                descs[j % 2] = pltpu.async_copy(
                    rb, agg_hbm.at[pl.ds(o0 + j * GB, GB)], (sem_g0, sem_g1)[j % 2])
            descs[0].wait()
            descs[1].wait()
            pltpu.sync_copy(acc.at[pl.ds(r0 + 768, 16)], rows0.at[pl.ds(0, 16)])
            pltpu.sync_copy(rows0.at[pl.ds(0, 16)], agg_hbm.at[pl.ds(o0 + 768, 16)])
            plsc.subcore_barrier()
            return carry

        lax.fori_loop(0, 2, _do_chunk, jnp.int32(0))


_agg_kernel = functools.partial(
    pl.kernel,
    compiler_params=pltpu.CompilerParams(needs_layout_passes=False),
    out_type=jax.ShapeDtypeStruct((AGG_ROWS, D), jnp.float32),
    mesh=plsc.VectorSubcoreMesh(core_axis_name="c", subcore_axis_name="s"),
    scratch_types=[
        pltpu.VMEM_SHARED((ACC_ROWS, D), jnp.float32),
        pltpu.VMEM((WIN,), jnp.int32),
        pltpu.VMEM((WIN,), jnp.int32),
        pltpu.VMEM((WIN,), jnp.int32),
        pltpu.VMEM((WIN,), jnp.int32),
        pltpu.VMEM((CAP,), jnp.int32),
        pltpu.VMEM((CAP,), jnp.int32),
        pltpu.VMEM((GB,), jnp.int32),
        pltpu.VMEM((GB,), jnp.int32),
        pltpu.VMEM((GB,), jnp.int32),
        pltpu.VMEM((GB,), jnp.int32),
        pltpu.VMEM((GB, D), jnp.float32),
        pltpu.VMEM((GB, D), jnp.float32),
        pltpu.SemaphoreType.DMA,
        pltpu.SemaphoreType.DMA,
        pltpu.SemaphoreType.DMA,
        pltpu.SemaphoreType.DMA,
        pltpu.SemaphoreType.DMA,
        pltpu.SemaphoreType.DMA,
    ],
)(_agg_body)


def _prescale_body(xu, xi, h, xsu, xsi):
    d0 = h[:, 0] + h[:, 1]
    xsu[...] = xu[...] * lax.rsqrt(jnp.clip(d0, 1.0, None))[:, None]
    d2 = h[:, 4] + h[:, 5]
    xsi[...] = xi[...] * lax.rsqrt(jnp.clip(d2, 1.0, None))[:, None]


def _prescale(x_user, x_item, hists_t):
    return pl.pallas_call(
        _prescale_body,
        grid=(N // BLK,),
        in_specs=[
            pl.BlockSpec((BLK, D), lambda i: (i, 0)),
            pl.BlockSpec((BLK, D), lambda i: (i, 0)),
            pl.BlockSpec((BLK, 8), lambda i: (i, 0)),
        ],
        out_specs=[
            pl.BlockSpec((BLK, D), lambda i: (i, 0)),
            pl.BlockSpec((BLK, D), lambda i: (i, 0)),
        ],
        out_shape=[
            jax.ShapeDtypeStruct((N, D), jnp.float32),
            jax.ShapeDtypeStruct((N, D), jnp.float32),
        ],
    )(x_user, x_item, hists_t)


def _out_body(agg, hd, w, b, out):
    nb = N // BLK
    is_user = pl.program_id(0) < nb
    h = hd[...]
    deg = jnp.where(is_user, h[:, 6] + h[:, 7], h[:, 2] + h[:, 3])
    nd = lax.rsqrt(jnp.clip(deg, 1.0, None))
    out[...] = (
        jnp.dot(agg[...] * nd[:, None], w[0], preferred_element_type=jnp.float32)
        + b[0, 0][None, :]
    )


def _finalize(agg, hists_t, w_st, b_st):
    nb = N // BLK
    return pl.pallas_call(
        _out_body,
        grid=(2 * nb,),
        in_specs=[
            pl.BlockSpec((BLK, D), lambda i: (jnp.where(i < nb, i, i + 1), 0)),
            pl.BlockSpec((BLK, 8), lambda i: (lax.rem(i, nb), 0)),
            pl.BlockSpec((1, D, D), lambda i: (i // nb, 0, 0)),
            pl.BlockSpec((1, 1, D), lambda i: (i // nb, 0, 0)),
        ],
        out_specs=pl.BlockSpec((BLK, D), lambda i: (i, 0)),
        out_shape=jax.ShapeDtypeStruct((2 * N, D), jnp.float32),
    )(agg, hists_t, w_st, b_st)


def kernel(x_user, x_item, edge_index_ui, edge_index_iu, W_buys, b_buys, W_rev, b_rev):
    su, du = edge_index_ui[0], edge_index_ui[1]
    si, di = edge_index_iu[0], edge_index_iu[1]

    pad_a = EPA - E
    sent = (N + (jnp.arange(pad_a, dtype=jnp.int32) % 128)).astype(jnp.int32)
    ids_a = jnp.concatenate([
        jnp.concatenate([su, sent]),
        jnp.concatenate([du, sent]),
        jnp.concatenate([si, sent]),
        jnp.concatenate([di, sent]),
    ])

    pad_c = EPC - E
    zpad = jnp.zeros((pad_c,), jnp.int32)
    bpad = jnp.full((pad_c,), BIG, jnp.int32)
    s0 = jnp.concatenate([si, zpad])
    d0 = jnp.concatenate([di, bpad])
    s1 = jnp.concatenate([su, zpad])
    d1 = jnp.concatenate([du, bpad])

    zeros_a = jnp.zeros((2048,), jnp.float32)
    zrows = jnp.zeros((GB, D), jnp.float32)

    hist_p = _hist_kernel(ids_a, zeros_a).reshape(NC, 4, HP)
    hists_t = jnp.transpose(hist_p, (1, 0, 2))[:, :, :N].reshape(8, N).T

    xs_user, xs_item = _prescale(x_user, x_item, hists_t)
    agg = _agg_kernel(xs_item, xs_user, s0, d0, s1, d1, zrows)

    w_st = jnp.stack([W_rev, W_buys])
    b_st = jnp.stack([b_rev, b_buys])[:, None, :]
    return _finalize(agg, hists_t, w_st, b_st)
